# baseline (speedup 1.0000x reference)
import functools

import jax
import jax.numpy as jnp
from jax.experimental import pallas as pl
from jax.experimental.pallas import tpu as pltpu

def _gating_kernel(x_ref, wg_ref, bg_ref, idx_ref, w_ref, xsum_ref, *, n_s, seq):
    s = pl.program_id(0)

    @pl.when(s == 0)
    def _():
        xsum_ref[...] = jnp.zeros_like(xsum_ref)

    xsum_ref[...] += jnp.sum(x_ref[...], axis=1)

    @pl.when(s == n_s - 1)
    def _():
        xm = xsum_ref[...] / seq
        logits = jax.lax.dot_general(
            xm, wg_ref[...], (((1,), (1,)), ((), ())),
            preferred_element_type=jnp.float32) + bg_ref[...]
        e = logits.shape[1]
        ids = jax.lax.broadcasted_iota(jnp.int32, logits.shape, 1)
        m1 = jnp.max(logits, axis=1, keepdims=True)
        i1 = jnp.min(jnp.where(logits == m1, ids, e), axis=1, keepdims=True)
        rest = jnp.where(ids == i1, -jnp.inf, logits)
        m2 = jnp.max(rest, axis=1, keepdims=True)
        i2 = jnp.min(jnp.where(rest == m2, ids, e), axis=1, keepdims=True)
        idx_ref[...] = jnp.concatenate([i1, i2], axis=1)
        e2 = jnp.exp(m2 - m1)
        denom = 1.0 + e2
        w_ref[...] = jnp.concatenate([1.0 / denom, e2 / denom], axis=1)


def _gelu_exact(v):
    return 0.5 * v * (1.0 + jax.lax.erf(v * 0.7071067811865476))


def _expert_kernel(idx_ref, w_ref, x_ref, w0_ref, w1_ref, b0_ref, b1_ref,
                   out_ref):
    b = pl.program_id(0)
    xb = x_ref[0]
    dn = (((1,), (1,)), ((), ()))
    y0 = jax.lax.dot_general(xb, w0_ref[0], dn,
                             preferred_element_type=jnp.float32)
    y0 = _gelu_exact(y0 + b0_ref[0])
    y1 = jax.lax.dot_general(xb, w1_ref[0], dn,
                             preferred_element_type=jnp.float32)
    y1 = _gelu_exact(y1 + b1_ref[0])
    out_ref[0] = w_ref[b, 0] * y0 + w_ref[b, 1] * y1


def kernel(x, Wg, bg, Wexp, bexp):
    b_sz, seq, d = x.shape
    e, o, _ = Wexp.shape
    k = 2

    bs_g = 2048
    n_sg = seq // bs_g
    idx, w = pl.pallas_call(
        functools.partial(_gating_kernel, n_s=n_sg, seq=seq),
        grid=(n_sg,),
        in_specs=[
            pl.BlockSpec((b_sz, bs_g, d), lambda s: (0, s, 0)),
            pl.BlockSpec((e, d), lambda s: (0, 0)),
            pl.BlockSpec((1, e), lambda s: (0, 0)),
        ],
        out_specs=[
            pl.BlockSpec((b_sz, k), lambda s: (0, 0)),
            pl.BlockSpec((b_sz, k), lambda s: (0, 0)),
        ],
        out_shape=[
            jax.ShapeDtypeStruct((b_sz, k), jnp.int32),
            jax.ShapeDtypeStruct((b_sz, k), jnp.float32),
        ],
        scratch_shapes=[pltpu.VMEM((b_sz, d), jnp.float32)],
    )(x, Wg, bg.reshape(1, e))

    bs = 1024
    n_s = seq // bs
    grid_spec = pltpu.PrefetchScalarGridSpec(
        num_scalar_prefetch=2,
        grid=(b_sz, n_s),
        in_specs=[
            pl.BlockSpec((1, bs, d), lambda b, s, idx, w: (b, s, 0)),
            pl.BlockSpec((1, o, d), lambda b, s, idx, w: (idx[b, 0], 0, 0)),
            pl.BlockSpec((1, o, d), lambda b, s, idx, w: (idx[b, 1], 0, 0)),
            pl.BlockSpec((1, 1, o), lambda b, s, idx, w: (idx[b, 0], 0, 0)),
            pl.BlockSpec((1, 1, o), lambda b, s, idx, w: (idx[b, 1], 0, 0)),
        ],
        out_specs=pl.BlockSpec((1, bs, o), lambda b, s, idx, w: (b, s, 0)),
    )
    out = pl.pallas_call(
        _expert_kernel,
        grid_spec=grid_spec,
        out_shape=jax.ShapeDtypeStruct((b_sz, seq, o), jnp.float32),
    )(idx, w, x, Wexp, Wexp, bexp.reshape(e, 1, o), bexp.reshape(e, 1, o))
    return out


# speedup vs baseline: 1.0072x; 1.0072x over previous
import functools

import jax
import jax.numpy as jnp
from jax.experimental import pallas as pl
from jax.experimental.pallas import tpu as pltpu

def _gating_kernel(x_ref, wg_ref, bg_ref, idx_ref, w_ref, xsum_ref, *, n_s, seq):
    s = pl.program_id(0)

    @pl.when(s == 0)
    def _():
        xsum_ref[...] = jnp.zeros_like(xsum_ref)

    xsum_ref[...] += jnp.sum(x_ref[...], axis=1)

    @pl.when(s == n_s - 1)
    def _():
        xm = xsum_ref[...] / seq
        logits = jax.lax.dot_general(
            xm, wg_ref[...], (((1,), (1,)), ((), ())),
            preferred_element_type=jnp.float32) + bg_ref[...]
        e = logits.shape[1]
        ids = jax.lax.broadcasted_iota(jnp.int32, logits.shape, 1)
        m1 = jnp.max(logits, axis=1, keepdims=True)
        i1 = jnp.min(jnp.where(logits == m1, ids, e), axis=1, keepdims=True)
        rest = jnp.where(ids == i1, -jnp.inf, logits)
        m2 = jnp.max(rest, axis=1, keepdims=True)
        i2 = jnp.min(jnp.where(rest == m2, ids, e), axis=1, keepdims=True)
        idx_ref[...] = jnp.concatenate([i1, i2], axis=1)
        e2 = jnp.exp(m2 - m1)
        denom = 1.0 + e2
        w_ref[...] = jnp.concatenate([1.0 / denom, e2 / denom], axis=1)


def _gelu_exact(v):
    return 0.5 * v * (1.0 + jax.lax.erf(v * 0.7071067811865476))


def _expert_kernel(idx_ref, w_ref, x_ref, w0_ref, w1_ref, b0_ref, b1_ref,
                   out_ref):
    b = pl.program_id(0)
    xb = x_ref[0]
    dn = (((1,), (1,)), ((), ()))
    y0 = jax.lax.dot_general(xb, w0_ref[0], dn,
                             preferred_element_type=jnp.float32)
    y1 = jax.lax.dot_general(xb, w1_ref[0], dn,
                             preferred_element_type=jnp.float32)
    g0 = _gelu_exact(y0 + b0_ref[0])
    g1 = _gelu_exact(y1 + b1_ref[0])
    out_ref[0] = w_ref[b, 0] * g0 + w_ref[b, 1] * g1


def kernel(x, Wg, bg, Wexp, bexp):
    b_sz, seq, d = x.shape
    e, o, _ = Wexp.shape
    k = 2

    bs_g = 1024
    n_sg = seq // bs_g
    idx, w = pl.pallas_call(
        functools.partial(_gating_kernel, n_s=n_sg, seq=seq),
        grid=(n_sg,),
        in_specs=[
            pl.BlockSpec((b_sz, bs_g, d), lambda s: (0, s, 0)),
            pl.BlockSpec((e, d), lambda s: (0, 0)),
            pl.BlockSpec((1, e), lambda s: (0, 0)),
        ],
        out_specs=[
            pl.BlockSpec((b_sz, k), lambda s: (0, 0)),
            pl.BlockSpec((b_sz, k), lambda s: (0, 0)),
        ],
        out_shape=[
            jax.ShapeDtypeStruct((b_sz, k), jnp.int32),
            jax.ShapeDtypeStruct((b_sz, k), jnp.float32),
        ],
        scratch_shapes=[pltpu.VMEM((b_sz, d), jnp.float32)],
    )(x, Wg, bg.reshape(1, e))

    bs = 1024
    n_s = seq // bs
    grid_spec = pltpu.PrefetchScalarGridSpec(
        num_scalar_prefetch=2,
        grid=(b_sz, n_s),
        in_specs=[
            pl.BlockSpec((1, bs, d), lambda b, s, idx, w: (b, s, 0)),
            pl.BlockSpec((1, o, d), lambda b, s, idx, w: (idx[b, 0], 0, 0)),
            pl.BlockSpec((1, o, d), lambda b, s, idx, w: (idx[b, 1], 0, 0)),
            pl.BlockSpec((1, 1, o), lambda b, s, idx, w: (idx[b, 0], 0, 0)),
            pl.BlockSpec((1, 1, o), lambda b, s, idx, w: (idx[b, 1], 0, 0)),
        ],
        out_specs=pl.BlockSpec((1, bs, o), lambda b, s, idx, w: (b, s, 0)),
    )
    out = pl.pallas_call(
        _expert_kernel,
        grid_spec=grid_spec,
        out_shape=jax.ShapeDtypeStruct((b_sz, seq, o), jnp.float32),
    )(idx, w, x, Wexp, Wexp, bexp.reshape(e, 1, o), bexp.reshape(e, 1, o))
    return out
